# Initial kernel scaffold; baseline (speedup 1.0000x reference)
import jax
import jax.numpy as jnp
from jax.experimental import pallas as pl


def kernel(seq, ts, node2vec, time2vec):
    raise NotImplementedError("write your pallas kernel here")


# speedup vs baseline: 3.5911x; 3.5911x over previous
import functools

import jax
import jax.numpy as jnp
from jax import lax
from jax.experimental import pallas as pl
from jax.experimental.pallas import tpu as pltpu
from jax.experimental.pallas import tpu_sc as plsc

D = 64
B = 4096
L = 200
TOTAL = B * L
CHUNK = 128
NROWS = TOTAL // CHUNK
NW = 32
ROWS_PER_W = NROWS // NW
G = 8
NGROUPS = ROWS_PER_W // G

_mesh = plsc.VectorSubcoreMesh(core_axis_name="c", subcore_axis_name="s")


@functools.partial(
    pl.kernel,
    mesh=_mesh,
    out_type=(
        jax.ShapeDtypeStruct((NROWS, CHUNK, D), jnp.float32),
        jax.ShapeDtypeStruct((NROWS, CHUNK, D), jnp.float32),
    ),
    scratch_types=[
        pltpu.VMEM((G, CHUNK), jnp.int32),
        pltpu.VMEM((CHUNK, D), jnp.float32),
        pltpu.SemaphoreType.DMA,
    ],
    compiler_params=pltpu.CompilerParams(use_tc_tiling_on_sc=False),
)
def _embed2(n2v, t2v, seq_i, ts_i, out_x, out_t, idx_v, rows_v, sem):
    wid = lax.axis_index("s") * 2 + lax.axis_index("c")
    row0 = wid * ROWS_PER_W
    for table, idx_hbm, out_hbm in ((n2v, seq_i, out_x), (t2v, ts_i, out_t)):
        def group_body(g, _, table=table, idx_hbm=idx_hbm, out_hbm=out_hbm):
            r0 = row0 + g * G
            pltpu.sync_copy(idx_hbm.at[pl.ds(r0, G)], idx_v)
            for b in range(G):
                pltpu.async_copy(table.at[idx_v.at[b]], rows_v, sem).wait()
                pltpu.sync_copy(rows_v, out_hbm.at[r0 + b])
            return ()
        lax.fori_loop(0, NGROUPS, group_body, ())


def kernel(seq, ts, node2vec, time2vec):
    seq_r = seq.reshape(NROWS, CHUNK).astype(jnp.int32)
    ts_r = ts.reshape(NROWS, CHUNK).astype(jnp.int32)
    x, t = _embed2(node2vec, time2vec, seq_r, ts_r)
    return x.reshape(B, L, D), t.reshape(B, L, D)


# speedup vs baseline: 4.0507x; 1.1280x over previous
import functools

import jax
import jax.numpy as jnp
from jax import lax
from jax.experimental import pallas as pl
from jax.experimental.pallas import tpu as pltpu
from jax.experimental.pallas import tpu_sc as plsc

D = 64
B = 4096
L = 200
TOTAL = B * L
CHUNK = 128
NROWS = TOTAL // CHUNK
NW = 32
ROWS_PER_W = NROWS // NW
NBUF = 2

_mesh = plsc.VectorSubcoreMesh(core_axis_name="c", subcore_axis_name="s")


@functools.partial(
    pl.kernel,
    mesh=_mesh,
    out_type=(
        jax.ShapeDtypeStruct((NROWS, CHUNK, D), jnp.float32),
        jax.ShapeDtypeStruct((NROWS, CHUNK, D), jnp.float32),
    ),
    scratch_types=[
        pltpu.VMEM((ROWS_PER_W, CHUNK), jnp.int32),
        pltpu.VMEM((NBUF, CHUNK, D), jnp.float32),
        pltpu.SemaphoreType.DMA,
        pltpu.SemaphoreType.DMA,
    ],
    compiler_params=pltpu.CompilerParams(use_tc_tiling_on_sc=False),
)
def _embed2(n2v, t2v, seq_i, ts_i, out_x, out_t, idx_v, rows, g0, g1):
    wid = lax.axis_index("s") * 2 + lax.axis_index("c")
    row0 = wid * ROWS_PER_W
    gsems = (g0, g1)
    for table, idx_hbm, out_hbm in ((n2v, seq_i, out_x), (t2v, ts_i, out_t)):
        pltpu.sync_copy(idx_hbm.at[pl.ds(row0, ROWS_PER_W)], idx_v)
        pltpu.async_copy(table.at[idx_v.at[0]], rows.at[0], gsems[0])
        def body(g, _, table=table, out_hbm=out_hbm):
            for b in range(NBUF):
                c = g * NBUF + b
                nxt = jnp.minimum(c + 1, ROWS_PER_W - 1)
                nb = (b + 1) % NBUF
                pltpu.async_copy(table.at[idx_v.at[nxt]], rows.at[nb], gsems[nb])
                pltpu.make_async_copy(table.at[idx_v.at[c]], rows.at[b], gsems[b]).wait()
                pltpu.sync_copy(rows.at[b], out_hbm.at[row0 + c])
            return ()
        lax.fori_loop(0, ROWS_PER_W // NBUF, body, ())
        pltpu.make_async_copy(table.at[idx_v.at[0]], rows.at[0], gsems[0]).wait()


def kernel(seq, ts, node2vec, time2vec):
    seq_r = seq.reshape(NROWS, CHUNK).astype(jnp.int32)
    ts_r = ts.reshape(NROWS, CHUNK).astype(jnp.int32)
    x, t = _embed2(node2vec, time2vec, seq_r, ts_r)
    return x.reshape(B, L, D), t.reshape(B, L, D)


# speedup vs baseline: 4.1331x; 1.0203x over previous
import functools

import jax
import jax.numpy as jnp
from jax import lax
from jax.experimental import pallas as pl
from jax.experimental.pallas import tpu as pltpu
from jax.experimental.pallas import tpu_sc as plsc

D = 64
B = 4096
L = 200
TOTAL = B * L
CHUNK = 128
NROWS = TOTAL // CHUNK
NW = 32
ROWS_PER_W = NROWS // NW
NBUF = 4
K = 3

_mesh = plsc.VectorSubcoreMesh(core_axis_name="c", subcore_axis_name="s")


@functools.partial(
    pl.kernel,
    mesh=_mesh,
    out_type=(
        jax.ShapeDtypeStruct((NROWS, CHUNK, D), jnp.float32),
        jax.ShapeDtypeStruct((NROWS, CHUNK, D), jnp.float32),
    ),
    scratch_types=[
        pltpu.VMEM((ROWS_PER_W, CHUNK), jnp.int32),
        pltpu.VMEM((NBUF, CHUNK, D), jnp.float32),
    ]
    + [pltpu.SemaphoreType.DMA] * (2 * NBUF),
    compiler_params=pltpu.CompilerParams(use_tc_tiling_on_sc=False),
)
def _embed2(n2v, t2v, seq_i, ts_i, out_x, out_t, idx_v, rows, *sems):
    wid = lax.axis_index("s") * 2 + lax.axis_index("c")
    row0 = wid * ROWS_PER_W
    gs, os_ = sems[:NBUF], sems[NBUF:]
    for table, idx_hbm, out_hbm in ((n2v, seq_i, out_x), (t2v, ts_i, out_t)):
        pltpu.sync_copy(idx_hbm.at[pl.ds(row0, ROWS_PER_W)], idx_v)
        for b in range(K):
            pltpu.async_copy(table.at[idx_v.at[b]], rows.at[b], gs[b])

        def body(g, _, table=table, out_hbm=out_hbm):
            for b in range(NBUF):
                c = g * NBUF + b
                pltpu.make_async_copy(table.at[idx_v.at[c]], rows.at[b], gs[b]).wait()
                pltpu.async_copy(rows.at[b], out_hbm.at[row0 + c], os_[b])
                nb = (b + K) % NBUF
                def recycle(c=c, nb=nb, table=table, out_hbm=out_hbm):
                    pltpu.make_async_copy(
                        rows.at[nb], out_hbm.at[row0 + c - 1], os_[nb]
                    ).wait()
                if b == 0:
                    pl.when(g > 0)(recycle)
                else:
                    recycle()
                nxt = jnp.minimum(c + K, ROWS_PER_W - 1)
                pltpu.async_copy(table.at[idx_v.at[nxt]], rows.at[nb], gs[nb])
            return ()

        lax.fori_loop(0, ROWS_PER_W // NBUF, body, ())
        for b in range(K):
            pltpu.make_async_copy(
                table.at[idx_v.at[ROWS_PER_W - 1]], rows.at[b], gs[b]
            ).wait()
        pltpu.make_async_copy(
            rows.at[NBUF - 1], out_hbm.at[row0 + ROWS_PER_W - 1], os_[NBUF - 1]
        ).wait()


def kernel(seq, ts, node2vec, time2vec):
    seq_r = seq.reshape(NROWS, CHUNK).astype(jnp.int32)
    ts_r = ts.reshape(NROWS, CHUNK).astype(jnp.int32)
    x, t = _embed2(node2vec, time2vec, seq_r, ts_r)
    return x.reshape(B, L, D), t.reshape(B, L, D)
